# Initial kernel scaffold; baseline (speedup 1.0000x reference)
import functools

import jax
import jax.numpy as jnp
from jax import lax
from jax.experimental import pallas as pl
from jax.experimental.pallas import tpu as pltpu
from jax.experimental.pallas import tpu_sc as plsc

_N = 10000
_E = 320000
_D = 128
_H = 128
_C = 40
_CP = 48

_NP = 10240
_ROWS_PER_TILE = _NP // 16

_K = 128
_CH = 80
_EP = 32 * _CH * _K

_BLK = 512


def _make_sc_propagate(width):
    mesh = plsc.VectorSubcoreMesh(core_axis_name="c", subcore_axis_name="s")

    @functools.partial(
        pl.kernel,
        mesh=mesh,
        out_type=jax.ShapeDtypeStruct((2, _NP, width), jnp.float32),
        scratch_types=[
            pltpu.VMEM((_CH, _K), jnp.int32),
            pltpu.VMEM((_CH, _K), jnp.int32),
            pltpu.VMEM((_K, width), jnp.float32),
            pltpu.VMEM((_K, width), jnp.float32),
            pltpu.VMEM_SHARED((_NP, width), jnp.float32),
            pltpu.SemaphoreType.DMA,
            pltpu.SemaphoreType.DMA,
        ],
    )
    def sc_fn(y_hbm, src_hbm, dst_hbm, zeros_hbm, out_hbm,
              src_v, dst_v, rows0, rows1, acc, sem0, sem1):
        cid = lax.axis_index("c")
        sid = lax.axis_index("s")
        gid = cid * 16 + sid
        row0 = sid * _ROWS_PER_TILE

        pltpu.sync_copy(zeros_hbm, acc.at[pl.ds(row0, _ROWS_PER_TILE)])
        pltpu.sync_copy(src_hbm.at[gid], src_v)
        pltpu.sync_copy(dst_hbm.at[gid], dst_v)
        plsc.subcore_barrier()

        pltpu.async_copy(y_hbm.at[src_v.at[0]], rows0, sem0)
        pltpu.async_copy(y_hbm.at[src_v.at[1]], rows1, sem1)

        def body(g, carry):
            c0 = 2 * g
            pltpu.make_async_copy(y_hbm.at[src_v.at[0]], rows0, sem0).wait()
            pltpu.sync_copy(rows0, acc.at[dst_v.at[c0]], add=True)

            @pl.when(g < _CH // 2 - 1)
            def _():
                pltpu.async_copy(y_hbm.at[src_v.at[c0 + 2]], rows0, sem0)

            pltpu.make_async_copy(y_hbm.at[src_v.at[1]], rows1, sem1).wait()
            pltpu.sync_copy(rows1, acc.at[dst_v.at[c0 + 1]], add=True)

            @pl.when(g < _CH // 2 - 1)
            def _():
                pltpu.async_copy(y_hbm.at[src_v.at[c0 + 3]], rows1, sem1)

            return carry

        lax.fori_loop(0, _CH // 2, body, 0)
        plsc.subcore_barrier()
        pltpu.sync_copy(acc.at[pl.ds(row0, _ROWS_PER_TILE)],
                        out_hbm.at[cid, pl.ds(row0, _ROWS_PER_TILE)])

    return sc_fn


_sc_prop_h = _make_sc_propagate(_H)
_sc_prop_c = _make_sc_propagate(_CP)


def _mm_first_body(x_ref, w_ref, o_ref):
    o_ref[...] = lax.dot_general(
        x_ref[...], w_ref[...], (((1,), (0,)), ((), ())),
        precision=lax.Precision.HIGHEST, preferred_element_type=jnp.float32)


def _mm_mid_body(p0_ref, p1_ref, b_ref, w_ref, o_ref):
    h = jnp.maximum(p0_ref[...] + p1_ref[...] + b_ref[...], 0.0)
    o_ref[...] = lax.dot_general(
        h, w_ref[...], (((1,), (0,)), ((), ())),
        precision=lax.Precision.HIGHEST, preferred_element_type=jnp.float32)


def _relu_body(p0_ref, p1_ref, b_ref, o_ref):
    o_ref[...] = jnp.maximum(p0_ref[...] + p1_ref[...] + b_ref[...], 0.0)


def _mm_first(x, w):
    din, dout = w.shape
    return pl.pallas_call(
        _mm_first_body,
        grid=(_NP // _BLK,),
        in_specs=[
            pl.BlockSpec((_BLK, din), lambda i: (i, 0)),
            pl.BlockSpec((din, dout), lambda i: (0, 0)),
        ],
        out_specs=pl.BlockSpec((_BLK, dout), lambda i: (i, 0)),
        out_shape=jax.ShapeDtypeStruct((_NP, dout), jnp.float32),
    )(x, w)


def _mm_mid(p0, p1, b, w):
    din, dout = w.shape
    return pl.pallas_call(
        _mm_mid_body,
        grid=(_NP // _BLK,),
        in_specs=[
            pl.BlockSpec((_BLK, din), lambda i: (i, 0)),
            pl.BlockSpec((_BLK, din), lambda i: (i, 0)),
            pl.BlockSpec((1, din), lambda i: (0, 0)),
            pl.BlockSpec((din, dout), lambda i: (0, 0)),
        ],
        out_specs=pl.BlockSpec((_BLK, dout), lambda i: (i, 0)),
        out_shape=jax.ShapeDtypeStruct((_NP, dout), jnp.float32),
    )(p0, p1, b, w)


def _relu_out(p0, p1, b):
    din = p0.shape[-1]
    return pl.pallas_call(
        _relu_body,
        grid=(_NP // _BLK,),
        in_specs=[
            pl.BlockSpec((_BLK, din), lambda i: (i, 0)),
            pl.BlockSpec((_BLK, din), lambda i: (i, 0)),
            pl.BlockSpec((1, din), lambda i: (0, 0)),
        ],
        out_specs=pl.BlockSpec((_BLK, din), lambda i: (i, 0)),
        out_shape=jax.ShapeDtypeStruct((_NP, din), jnp.float32),
    )(p0, p1, b)


def kernel(features, edge_index, W1, b1, W2, b2, W3, b3):
    f = jnp.pad(features, ((0, _NP - _N), (0, 0)))
    src = jnp.pad(edge_index[0], (0, _EP - _E)).reshape(32, _CH, _K)
    dst = jnp.pad(edge_index[1], (0, _EP - _E),
                  constant_values=_N).reshape(32, _CH, _K)
    w3p = jnp.pad(W3, ((0, 0), (0, _CP - _C)))
    b3p = jnp.pad(b3, (0, _CP - _C)).reshape(1, _CP)
    zh = jnp.zeros((_ROWS_PER_TILE, _H), jnp.float32)
    zc = jnp.zeros((_ROWS_PER_TILE, _CP), jnp.float32)

    y1 = _mm_first(f, W1)
    p1 = _sc_prop_h(y1, src, dst, zh)
    y2 = _mm_mid(p1[0], p1[1], b1.reshape(1, _H), W2)
    p2 = _sc_prop_h(y2, src, dst, zh)
    y3 = _mm_mid(p2[0], p2[1], b2.reshape(1, _H), w3p)
    p3 = _sc_prop_c(y3, src, dst, zc)
    out = _relu_out(p3[0], p3[1], b3p)
    return out[:_N, :_C]


# speedup vs baseline: 2.0227x; 2.0227x over previous
import functools

import jax
import jax.numpy as jnp
from jax import lax
from jax.experimental import pallas as pl
from jax.experimental.pallas import tpu as pltpu
from jax.experimental.pallas import tpu_sc as plsc

_N = 10000
_E = 320000
_D = 128
_C = 40

_NP = 10240
_HALF = _NP // 2
_R = 5248
_RT = _R // 16
_DUMMY = _HALF

_K = 128
_CH = 160
_EP = 16 * _CH * _K

_BLK = 512


_sc_mesh = plsc.VectorSubcoreMesh(core_axis_name="c", subcore_axis_name="s")


@functools.partial(
    pl.kernel,
    mesh=_sc_mesh,
    out_type=jax.ShapeDtypeStruct((2, _R, _D), jnp.float32),
    scratch_types=[
        pltpu.VMEM((_CH, _K), jnp.int32),
        pltpu.VMEM((_CH, _K), jnp.int32),
        pltpu.VMEM((_K, _D), jnp.float32),
        pltpu.VMEM((_K, _D), jnp.float32),
        pltpu.VMEM_SHARED((_R, _D), jnp.float32),
        pltpu.SemaphoreType.DMA,
        pltpu.SemaphoreType.DMA,
    ],
)
def _sc_propagate(y_hbm, src_hbm, dst_hbm, zeros_hbm, out_hbm,
                  src_v, dst_v, rows0, rows1, acc, sem0, sem1):
    cid = lax.axis_index("c")
    sid = lax.axis_index("s")
    row0 = sid * _RT

    pltpu.sync_copy(src_hbm.at[sid], src_v)
    pltpu.sync_copy(dst_hbm.at[cid, sid], dst_v)
    pltpu.sync_copy(zeros_hbm, acc.at[pl.ds(row0, _RT)])
    plsc.subcore_barrier()

    pltpu.async_copy(y_hbm.at[src_v.at[0]], rows0, sem0)
    pltpu.async_copy(y_hbm.at[src_v.at[1]], rows1, sem1)

    def body(g, carry):
        c0 = 2 * g
        pltpu.make_async_copy(y_hbm.at[src_v.at[0]], rows0, sem0).wait()
        pltpu.sync_copy(rows0, acc.at[dst_v.at[c0]], add=True)

        @pl.when(g < _CH // 2 - 1)
        def _():
            pltpu.async_copy(y_hbm.at[src_v.at[c0 + 2]], rows0, sem0)

        pltpu.make_async_copy(y_hbm.at[src_v.at[1]], rows1, sem1).wait()
        pltpu.sync_copy(rows1, acc.at[dst_v.at[c0 + 1]], add=True)

        @pl.when(g < _CH // 2 - 1)
        def _():
            pltpu.async_copy(y_hbm.at[src_v.at[c0 + 3]], rows1, sem1)

        return carry

    lax.fori_loop(0, _CH // 2, body, 0)
    plsc.subcore_barrier()
    pltpu.sync_copy(acc.at[pl.ds(row0, _RT)],
                    out_hbm.at[cid, pl.ds(row0, _RT)])


def _mm_first_body(x_ref, w_ref, o_ref):
    o_ref[...] = lax.dot_general(
        x_ref[...], w_ref[...], (((1,), (0,)), ((), ())),
        precision=lax.Precision.HIGHEST, preferred_element_type=jnp.float32)


def _mm_mid_body(p_ref, b_ref, w_ref, o_ref):
    h = jnp.maximum(p_ref[0] + b_ref[...], 0.0)
    o_ref[...] = lax.dot_general(
        h, w_ref[...], (((1,), (0,)), ((), ())),
        precision=lax.Precision.HIGHEST, preferred_element_type=jnp.float32)


def _relu_body(p_ref, b_ref, o_ref):
    o_ref[...] = jnp.maximum(p_ref[0] + b_ref[...], 0.0)


def _mm_first(x, w):
    return pl.pallas_call(
        _mm_first_body,
        grid=(_NP // _BLK,),
        in_specs=[
            pl.BlockSpec((_BLK, _D), lambda i: (i, 0)),
            pl.BlockSpec((_D, _D), lambda i: (0, 0)),
        ],
        out_specs=pl.BlockSpec((_BLK, _D), lambda i: (i, 0)),
        out_shape=jax.ShapeDtypeStruct((_NP, _D), jnp.float32),
    )(x, w)


def _mm_mid(p, b, w):
    return pl.pallas_call(
        _mm_mid_body,
        grid=(_NP // _BLK,),
        in_specs=[
            pl.BlockSpec((1, _BLK, _D), lambda i: (i // 10, i % 10, 0)),
            pl.BlockSpec((1, _D), lambda i: (0, 0)),
            pl.BlockSpec((_D, _D), lambda i: (0, 0)),
        ],
        out_specs=pl.BlockSpec((_BLK, _D), lambda i: (i, 0)),
        out_shape=jax.ShapeDtypeStruct((_NP, _D), jnp.float32),
    )(p, b.reshape(1, _D), w)


def _relu_out(p, b):
    return pl.pallas_call(
        _relu_body,
        grid=(_NP // _BLK,),
        in_specs=[
            pl.BlockSpec((1, _BLK, _D), lambda i: (i // 10, i % 10, 0)),
            pl.BlockSpec((1, _D), lambda i: (0, 0)),
        ],
        out_specs=pl.BlockSpec((_BLK, _D), lambda i: (i, 0)),
        out_shape=jax.ShapeDtypeStruct((_NP, _D), jnp.float32),
    )(p, b.reshape(1, _D))


def kernel(features, edge_index, W1, b1, W2, b2, W3, b3):
    f = jnp.pad(features, ((0, _NP - _N), (0, 0)))
    src = jnp.pad(edge_index[0], (0, _EP - _E)).reshape(16, _CH, _K)
    dstf = jnp.pad(edge_index[1], (0, _EP - _E), constant_values=_N)
    dst_cores = []
    for c in range(2):
        lo, hi = c * _HALF, (c + 1) * _HALF
        local = jnp.where((dstf >= lo) & (dstf < hi), dstf - lo, _DUMMY)
        dst_cores.append(local.astype(jnp.int32).reshape(16, _CH, _K))
    dst = jnp.stack(dst_cores)
    w3p = jnp.pad(W3, ((0, 0), (0, _D - _C)))
    b3p = jnp.pad(b3, (0, _D - _C))
    z = jnp.zeros((_RT, _D), jnp.float32)

    y1 = _mm_first(f, W1)
    p1 = _sc_propagate(y1, src, dst, z)
    y2 = _mm_mid(p1, b1, W2)
    p2 = _sc_propagate(y2, src, dst, z)
    y3 = _mm_mid(p2, b2, w3p)
    p3 = _sc_propagate(y3, src, dst, z)
    out = _relu_out(p3, b3p)
    return out[:_N, :_C]
